# baseline (speedup 1.0000x reference)
import functools
import math

import jax
import jax.numpy as jnp
from jax import lax
from jax.experimental import pallas as pl
from jax.experimental.pallas import tpu as pltpu
from jax.experimental.pallas import tpu_sc as plsc

_E = 8
_K = 2
_C = 96
_B = 4
_H = 56
_W = 56
_HP = _H + 2
_WP = _W + 2
_P = _HP * _WP
_MARGIN = _WP + 1
_XE = ((_P + 2 * _MARGIN + 127) // 128) * 128
_NOISE_STD = 0.1
_OFFS = tuple((dy - 1) * _WP + (dx - 1) for dy in range(3) for dx in range(3))
_D = 512
_L = 16


def _gate_sc_body(tf_hbm, wg_hbm, noise_hbm, idx_hbm, val_hbm,
                  tf_v, wg_v, noise_v, ibuf, vbuf):
    wid = lax.axis_index("s") * 2 + lax.axis_index("c")

    @pl.when(wid < _B)
    def _work():
        b = wid
        pltpu.sync_copy(tf_hbm.at[b], tf_v)
        pltpu.sync_copy(wg_hbm, wg_v)
        pltpu.sync_copy(noise_hbm.at[b], noise_v)

        lane = lax.iota(jnp.int32, _L)
        logits = jnp.full((_L,), -1e30, jnp.float32)
        for e in range(_E):
            acc = jnp.zeros((_L,), jnp.float32)
            for c in range(_D // _L):
                acc = acc + tf_v[pl.ds(c * _L, _L)] * wg_v[e, pl.ds(c * _L, _L)]
            s = jnp.sum(acc, axis=0)
            logits = jnp.where(lane == e, s, logits)
        logits = logits + noise_v[pl.ds(0, _L)]
        m = jnp.max(logits, axis=0)
        ex = jnp.exp(logits - m)
        w = ex / jnp.sum(ex, axis=0)
        w_sorted, i_sorted = plsc.sort_key_val(w, lane, descending=True)
        for c in range(128 // _L):
            vbuf[pl.ds(c * _L, _L)] = w_sorted
            ibuf[pl.ds(c * _L, _L)] = i_sorted
        pltpu.sync_copy(vbuf, val_hbm.at[b])
        pltpu.sync_copy(ibuf, idx_hbm.at[b])


def _gate_sc(text_feature, Wg, noise_eff):
    mesh = plsc.VectorSubcoreMesh(core_axis_name="c", subcore_axis_name="s")
    k = functools.partial(
        pl.kernel,
        mesh=mesh,
        out_type=(
            jax.ShapeDtypeStruct((_B, 128), jnp.int32),
            jax.ShapeDtypeStruct((_B, 128), jnp.float32),
        ),
        scratch_types=[
            pltpu.VMEM((_D,), jnp.float32),
            pltpu.VMEM((_E, _D), jnp.float32),
            pltpu.VMEM((128,), jnp.float32),
            pltpu.VMEM((128,), jnp.int32),
            pltpu.VMEM((128,), jnp.float32),
        ],
        compiler_params=pltpu.CompilerParams(needs_layout_passes=False),
    )(_gate_sc_body)
    return k(text_feature, Wg, noise_eff)


def _gelu(x):
    return 0.5 * x * (1.0 + jax.lax.erf(x * (1.0 / math.sqrt(2.0))))


def _moe_body(idx_ref, val_ref, xp_ref, mask_ref, w1a_ref, w1b_ref,
              b1a_ref, b1b_ref, w2a_ref, w2b_ref, b2a_ref, b2b_ref,
              out_ref, xext, xcat, hext, hacat, hbcat):
    b = pl.program_id(0)
    s0 = val_ref[b, 0]
    s1 = val_ref[b, 1]

    @pl.when(b == 0)
    def _init():
        xext[...] = jnp.zeros((_C, _XE), jnp.bfloat16)
        hext[...] = jnp.zeros((2 * _C, _XE), jnp.bfloat16)

    xext[:, _MARGIN:_MARGIN + _P] = xp_ref[0]
    for t, o in enumerate(_OFFS):
        xcat[t * _C:(t + 1) * _C, :] = xext[:, _MARGIN + o:_MARGIN + o + _P]

    y1a = jax.lax.dot_general(
        w1a_ref[0], xcat[...], (((1,), (0,)), ((), ())),
        preferred_element_type=jnp.float32)
    y1b = jax.lax.dot_general(
        w1b_ref[0], xcat[...], (((1,), (0,)), ((), ())),
        preferred_element_type=jnp.float32)
    bias1 = jnp.concatenate([b1a_ref[0], b1b_ref[0]], axis=0)
    h = _gelu(jnp.concatenate([y1a, y1b], axis=0) + bias1)

    rows = jax.lax.broadcasted_iota(jnp.int32, (2 * _C, 1), 0)
    scale = jnp.where(rows < _C, s0, s1)
    h = h * mask_ref[0] * scale

    hext[:, _MARGIN:_MARGIN + _P] = h.astype(jnp.bfloat16)
    for t, o in enumerate(_OFFS):
        hacat[t * _C:(t + 1) * _C, :] = hext[0:_C, _MARGIN + o:_MARGIN + o + _P]
        hbcat[t * _C:(t + 1) * _C, :] = hext[_C:2 * _C, _MARGIN + o:_MARGIN + o + _P]

    y2 = jax.lax.dot_general(
        w2a_ref[0], hacat[...], (((1,), (0,)), ((), ())),
        preferred_element_type=jnp.float32)
    y2 = y2 + jax.lax.dot_general(
        w2b_ref[0], hbcat[...], (((1,), (0,)), ((), ())),
        preferred_element_type=jnp.float32)
    bias2 = s0 * b2a_ref[0] + s1 * b2b_ref[0]
    out_ref[0] = y2 + bias2


@jax.jit
def kernel(x, text_feature, training, Wg, W1, b1, W2, b2):
    B = x.shape[0]
    noise = jax.random.normal(jax.random.key(42), (B, _E), jnp.float32) * _NOISE_STD
    noise_eff = jnp.where(jnp.asarray(training) != 0, noise, 0.0)

    noise128 = jnp.zeros((B, 128), jnp.float32).at[:, :_E].set(noise_eff)
    idx_full, val_full = _gate_sc(text_feature, Wg, noise128)
    idx = idx_full[:, :_K]
    vals = val_full[:, :_K]

    xp = jnp.pad(x, ((0, 0), (0, 0), (1, 1), (1, 1)))
    xp = xp.reshape(B, _C, _P).astype(jnp.bfloat16)
    ii = jnp.arange(_P, dtype=jnp.int32) // _WP
    jj = jnp.arange(_P, dtype=jnp.int32) % _WP
    mask = ((ii >= 1) & (ii <= _H) & (jj >= 1) & (jj <= _W))
    mask = mask.astype(jnp.float32).reshape(1, 1, _P)
    W1t = W1.transpose(0, 1, 3, 4, 2).reshape(_E, _C, 9 * _C).astype(jnp.bfloat16)
    W2t = W2.transpose(0, 1, 3, 4, 2).reshape(_E, _C, 9 * _C).astype(jnp.bfloat16)
    b1r = b1.reshape(_E, _C, 1)
    b2r = b2.reshape(_E, _C, 1)

    wspec = lambda k: pl.BlockSpec((1, _C, 9 * _C),
                                   lambda b, idx, val, k=k: (idx[b, k], 0, 0))
    bspec = lambda k: pl.BlockSpec((1, _C, 1),
                                   lambda b, idx, val, k=k: (idx[b, k], 0, 0))
    grid_spec = pltpu.PrefetchScalarGridSpec(
        num_scalar_prefetch=2,
        grid=(B,),
        in_specs=[
            pl.BlockSpec((1, _C, _P), lambda b, idx, val: (b, 0, 0)),
            pl.BlockSpec((1, 1, _P), lambda b, idx, val: (0, 0, 0)),
            wspec(0), wspec(1), bspec(0), bspec(1),
            wspec(0), wspec(1), bspec(0), bspec(1),
        ],
        out_specs=pl.BlockSpec((1, _C, _P), lambda b, idx, val: (b, 0, 0)),
        scratch_shapes=[
            pltpu.VMEM((_C, _XE), jnp.bfloat16),
            pltpu.VMEM((9 * _C, _P), jnp.bfloat16),
            pltpu.VMEM((2 * _C, _XE), jnp.bfloat16),
            pltpu.VMEM((9 * _C, _P), jnp.bfloat16),
            pltpu.VMEM((9 * _C, _P), jnp.bfloat16),
        ],
    )
    out = pl.pallas_call(
        _moe_body,
        grid_spec=grid_spec,
        out_shape=jax.ShapeDtypeStruct((B, _C, _P), jnp.float32),
        compiler_params=pltpu.CompilerParams(
            dimension_semantics=("arbitrary",)),
    )(idx, vals, xp, mask, W1t, W1t, b1r, b1r, W2t, W2t, b2r, b2r)

    return out.reshape(B, _C, _HP, _WP)[:, :, 1:-1, 1:-1]


# speedup vs baseline: 1.2852x; 1.2852x over previous
import math

import jax
import jax.numpy as jnp
from jax.experimental import pallas as pl
from jax.experimental.pallas import tpu as pltpu

_E = 8
_K = 2
_C = 96
_B = 4
_H = 56
_W = 56
_HP = _H + 2
_WP = _W + 2
_P = _HP * _WP
_MARGIN = _WP + 1
_XE = ((_P + 2 * _MARGIN + 127) // 128) * 128
_NOISE_STD = 0.1
_OFFS = tuple((dy - 1) * _WP + (dx - 1) for dy in range(3) for dx in range(3))


def _gate_body(tf_ref, wg_ref, noise_ref, idx_ref, val_ref):
    t = tf_ref[...]
    wg = wg_ref[...]
    logits = jax.lax.dot_general(
        t, wg, (((1,), (1,)), ((), ())),
        preferred_element_type=jnp.float32)
    logits = logits + noise_ref[...]
    m = jnp.max(logits, axis=-1, keepdims=True)
    e = jnp.exp(logits - m)
    w = e / jnp.sum(e, axis=-1, keepdims=True)
    col = jax.lax.broadcasted_iota(jnp.int32, w.shape, 1)
    v0 = jnp.max(w, axis=-1, keepdims=True)
    i0 = jnp.min(jnp.where(w == v0, col, _E), axis=-1, keepdims=True)
    w2 = jnp.where(col == i0, -1.0, w)
    v1 = jnp.max(w2, axis=-1, keepdims=True)
    i1 = jnp.min(jnp.where(w2 == v1, col, _E), axis=-1, keepdims=True)
    k2 = jax.lax.broadcasted_iota(jnp.int32, (t.shape[0], _K), 1)
    idx_ref[...] = jnp.where(k2 == 0, i0, i1)
    val_ref[...] = jnp.where(k2 == 0, v0, v1)


def _gelu(x):
    return 0.5 * x * (1.0 + jax.lax.erf(x * (1.0 / math.sqrt(2.0))))


def _moe_body(idx_ref, val_ref, xp_ref, mask_ref, w1a_ref, w1b_ref,
              b1a_ref, b1b_ref, w2a_ref, w2b_ref, b2a_ref, b2b_ref,
              out_ref, xext, xcat, hext, hacat, hbcat):
    b = pl.program_id(0)
    s0 = val_ref[b, 0]
    s1 = val_ref[b, 1]

    @pl.when(b == 0)
    def _init():
        xext[...] = jnp.zeros((_C, _XE), jnp.bfloat16)
        hext[...] = jnp.zeros((2 * _C, _XE), jnp.bfloat16)

    xext[:, _MARGIN:_MARGIN + _P] = xp_ref[0]
    for t, o in enumerate(_OFFS):
        xcat[t * _C:(t + 1) * _C, :] = xext[:, _MARGIN + o:_MARGIN + o + _P]

    y1a = jax.lax.dot_general(
        w1a_ref[0], xcat[...], (((1,), (0,)), ((), ())),
        preferred_element_type=jnp.float32)
    y1b = jax.lax.dot_general(
        w1b_ref[0], xcat[...], (((1,), (0,)), ((), ())),
        preferred_element_type=jnp.float32)
    bias1 = jnp.concatenate([b1a_ref[0], b1b_ref[0]], axis=0)
    h = _gelu(jnp.concatenate([y1a, y1b], axis=0) + bias1)

    rows = jax.lax.broadcasted_iota(jnp.int32, (2 * _C, 1), 0)
    scale = jnp.where(rows < _C, s0, s1)
    h = h * mask_ref[0] * scale

    hext[:, _MARGIN:_MARGIN + _P] = h.astype(jnp.bfloat16)
    for t, o in enumerate(_OFFS):
        hacat[t * _C:(t + 1) * _C, :] = hext[0:_C, _MARGIN + o:_MARGIN + o + _P]
        hbcat[t * _C:(t + 1) * _C, :] = hext[_C:2 * _C, _MARGIN + o:_MARGIN + o + _P]

    y2 = jax.lax.dot_general(
        w2a_ref[0], hacat[...], (((1,), (0,)), ((), ())),
        preferred_element_type=jnp.float32)
    y2 = y2 + jax.lax.dot_general(
        w2b_ref[0], hbcat[...], (((1,), (0,)), ((), ())),
        preferred_element_type=jnp.float32)
    bias2 = s0 * b2a_ref[0] + s1 * b2b_ref[0]
    out_ref[0] = (y2 + bias2).astype(jnp.bfloat16)


@jax.jit
def kernel(x, text_feature, training, Wg, W1, b1, W2, b2):
    B = x.shape[0]
    noise = jax.random.normal(jax.random.key(42), (B, _E), jnp.float32) * _NOISE_STD
    noise_eff = jnp.where(jnp.asarray(training) != 0, noise, 0.0)

    idx, vals = pl.pallas_call(
        _gate_body,
        out_shape=(
            jax.ShapeDtypeStruct((B, _K), jnp.int32),
            jax.ShapeDtypeStruct((B, _K), jnp.float32),
        ),
    )(text_feature, Wg, noise_eff)

    xp = jnp.pad(x, ((0, 0), (0, 0), (1, 1), (1, 1)))
    xp = xp.reshape(B, _C, _P).astype(jnp.bfloat16)
    ii = jnp.arange(_P, dtype=jnp.int32) // _WP
    jj = jnp.arange(_P, dtype=jnp.int32) % _WP
    mask = ((ii >= 1) & (ii <= _H) & (jj >= 1) & (jj <= _W))
    mask = mask.astype(jnp.float32).reshape(1, 1, _P)
    W1t = W1.transpose(0, 1, 3, 4, 2).reshape(_E, _C, 9 * _C).astype(jnp.bfloat16)
    W2t = W2.transpose(0, 1, 3, 4, 2).reshape(_E, _C, 9 * _C).astype(jnp.bfloat16)
    b1r = b1.reshape(_E, _C, 1)
    b2r = b2.reshape(_E, _C, 1)

    wspec = lambda k: pl.BlockSpec((1, _C, 9 * _C),
                                   lambda b, idx, val, k=k: (idx[b, k], 0, 0))
    bspec = lambda k: pl.BlockSpec((1, _C, 1),
                                   lambda b, idx, val, k=k: (idx[b, k], 0, 0))
    grid_spec = pltpu.PrefetchScalarGridSpec(
        num_scalar_prefetch=2,
        grid=(B,),
        in_specs=[
            pl.BlockSpec((1, _C, _P), lambda b, idx, val: (b, 0, 0)),
            pl.BlockSpec((1, 1, _P), lambda b, idx, val: (0, 0, 0)),
            wspec(0), wspec(1), bspec(0), bspec(1),
            wspec(0), wspec(1), bspec(0), bspec(1),
        ],
        out_specs=pl.BlockSpec((1, _C, _P), lambda b, idx, val: (b, 0, 0)),
        scratch_shapes=[
            pltpu.VMEM((_C, _XE), jnp.bfloat16),
            pltpu.VMEM((9 * _C, _P), jnp.bfloat16),
            pltpu.VMEM((2 * _C, _XE), jnp.bfloat16),
            pltpu.VMEM((9 * _C, _P), jnp.bfloat16),
            pltpu.VMEM((9 * _C, _P), jnp.bfloat16),
        ],
    )
    out = pl.pallas_call(
        _moe_body,
        grid_spec=grid_spec,
        out_shape=jax.ShapeDtypeStruct((B, _C, _P), jnp.bfloat16),
        compiler_params=pltpu.CompilerParams(
            dimension_semantics=("arbitrary",)),
    )(idx, vals, xp, mask, W1t, W1t, b1r, b1r, W2t, W2t, b2r, b2r)

    out = out.reshape(B, _C, _HP, _WP)[:, :, 1:-1, 1:-1]
    return out.astype(jnp.float32)


# speedup vs baseline: 1.2924x; 1.0056x over previous
import math

import jax
import jax.numpy as jnp
from jax.experimental import pallas as pl
from jax.experimental.pallas import tpu as pltpu

_E = 8
_K = 2
_C = 96
_B = 4
_H = 56
_W = 56
_HP = _H + 2
_WP = _W + 2
_P = _HP * _WP
_MARGIN = _WP + 1
_XE = ((_P + 2 * _MARGIN + 127) // 128) * 128
_NOISE_STD = 0.1
_OFFS = tuple((dy - 1) * _WP + (dx - 1) for dy in range(3) for dx in range(3))


def _gate_body(tf_ref, wg_ref, noise_ref, idx_ref, val_ref):
    t = tf_ref[...]
    wg = wg_ref[...]
    logits = jax.lax.dot_general(
        t, wg, (((1,), (1,)), ((), ())),
        preferred_element_type=jnp.float32)
    logits = logits + noise_ref[...]
    m = jnp.max(logits, axis=-1, keepdims=True)
    e = jnp.exp(logits - m)
    w = e / jnp.sum(e, axis=-1, keepdims=True)
    col = jax.lax.broadcasted_iota(jnp.int32, w.shape, 1)
    v0 = jnp.max(w, axis=-1, keepdims=True)
    i0 = jnp.min(jnp.where(w == v0, col, _E), axis=-1, keepdims=True)
    w2 = jnp.where(col == i0, -1.0, w)
    v1 = jnp.max(w2, axis=-1, keepdims=True)
    i1 = jnp.min(jnp.where(w2 == v1, col, _E), axis=-1, keepdims=True)
    k2 = jax.lax.broadcasted_iota(jnp.int32, (t.shape[0], _K), 1)
    idx_ref[...] = jnp.where(k2 == 0, i0, i1)
    val_ref[...] = jnp.where(k2 == 0, v0, v1)


def _gelu(x):
    return 0.5 * x * (1.0 + jax.lax.erf(x * (1.0 / math.sqrt(2.0))))


def _moe_body(idx_ref, val_ref, xp_ref, mask_ref, w1a_ref, w1b_ref,
              b1a_ref, b1b_ref, w2a_ref, w2b_ref, b2a_ref, b2b_ref,
              out_ref, xext, xcat, hext, hacat, hbcat):
    b = pl.program_id(0)
    s0 = val_ref[b, 0]
    s1 = val_ref[b, 1]

    xext[:, 0:_MARGIN] = jnp.zeros((_C, _MARGIN), jnp.bfloat16)
    xext[:, _MARGIN + _P:_XE] = jnp.zeros((_C, _XE - _MARGIN - _P), jnp.bfloat16)
    hext[:, 0:_MARGIN] = jnp.zeros((2 * _C, _MARGIN), jnp.bfloat16)
    hext[:, _MARGIN + _P:_XE] = jnp.zeros((2 * _C, _XE - _MARGIN - _P), jnp.bfloat16)

    xext[:, _MARGIN:_MARGIN + _P] = xp_ref[0]
    for t, o in enumerate(_OFFS):
        xcat[t * _C:(t + 1) * _C, :] = xext[:, _MARGIN + o:_MARGIN + o + _P]

    y1a = jax.lax.dot_general(
        w1a_ref[0], xcat[...], (((1,), (0,)), ((), ())),
        preferred_element_type=jnp.float32)
    y1b = jax.lax.dot_general(
        w1b_ref[0], xcat[...], (((1,), (0,)), ((), ())),
        preferred_element_type=jnp.float32)
    bias1 = jnp.concatenate([b1a_ref[0], b1b_ref[0]], axis=0)
    h = _gelu(jnp.concatenate([y1a, y1b], axis=0) + bias1)

    rows = jax.lax.broadcasted_iota(jnp.int32, (2 * _C, 1), 0)
    scale = jnp.where(rows < _C, s0, s1)
    h = h * mask_ref[0] * scale

    hext[:, _MARGIN:_MARGIN + _P] = h.astype(jnp.bfloat16)
    for t, o in enumerate(_OFFS):
        hacat[t * _C:(t + 1) * _C, :] = hext[0:_C, _MARGIN + o:_MARGIN + o + _P]
        hbcat[t * _C:(t + 1) * _C, :] = hext[_C:2 * _C, _MARGIN + o:_MARGIN + o + _P]

    y2 = jax.lax.dot_general(
        w2a_ref[0], hacat[...], (((1,), (0,)), ((), ())),
        preferred_element_type=jnp.float32)
    y2 = y2 + jax.lax.dot_general(
        w2b_ref[0], hbcat[...], (((1,), (0,)), ((), ())),
        preferred_element_type=jnp.float32)
    bias2 = s0 * b2a_ref[0] + s1 * b2b_ref[0]
    out_ref[0] = (y2 + bias2).astype(jnp.bfloat16)


@jax.jit
def kernel(x, text_feature, training, Wg, W1, b1, W2, b2):
    B = x.shape[0]
    noise = jax.random.normal(jax.random.key(42), (B, _E), jnp.float32) * _NOISE_STD
    noise_eff = jnp.where(jnp.asarray(training) != 0, noise, 0.0)

    idx, vals = pl.pallas_call(
        _gate_body,
        out_shape=(
            jax.ShapeDtypeStruct((B, _K), jnp.int32),
            jax.ShapeDtypeStruct((B, _K), jnp.float32),
        ),
    )(text_feature, Wg, noise_eff)

    xp = jnp.pad(x, ((0, 0), (0, 0), (1, 1), (1, 1)))
    xp = xp.reshape(B, _C, _P).astype(jnp.bfloat16)
    ii = jnp.arange(_P, dtype=jnp.int32) // _WP
    jj = jnp.arange(_P, dtype=jnp.int32) % _WP
    mask = ((ii >= 1) & (ii <= _H) & (jj >= 1) & (jj <= _W))
    mask = mask.astype(jnp.float32).reshape(1, 1, _P)
    W1t = W1.transpose(0, 1, 3, 4, 2).reshape(_E, _C, 9 * _C).astype(jnp.bfloat16)
    W2t = W2.transpose(0, 1, 3, 4, 2).reshape(_E, _C, 9 * _C).astype(jnp.bfloat16)
    b1r = b1.reshape(_E, _C, 1)
    b2r = b2.reshape(_E, _C, 1)

    wspec = lambda k: pl.BlockSpec((1, _C, 9 * _C),
                                   lambda b, idx, val, k=k: (idx[b, k], 0, 0))
    bspec = lambda k: pl.BlockSpec((1, _C, 1),
                                   lambda b, idx, val, k=k: (idx[b, k], 0, 0))
    grid_spec = pltpu.PrefetchScalarGridSpec(
        num_scalar_prefetch=2,
        grid=(B,),
        in_specs=[
            pl.BlockSpec((1, _C, _P), lambda b, idx, val: (b, 0, 0)),
            pl.BlockSpec((1, 1, _P), lambda b, idx, val: (0, 0, 0)),
            wspec(0), wspec(1), bspec(0), bspec(1),
            wspec(0), wspec(1), bspec(0), bspec(1),
        ],
        out_specs=pl.BlockSpec((1, _C, _P), lambda b, idx, val: (b, 0, 0)),
        scratch_shapes=[
            pltpu.VMEM((_C, _XE), jnp.bfloat16),
            pltpu.VMEM((9 * _C, _P), jnp.bfloat16),
            pltpu.VMEM((2 * _C, _XE), jnp.bfloat16),
            pltpu.VMEM((9 * _C, _P), jnp.bfloat16),
            pltpu.VMEM((9 * _C, _P), jnp.bfloat16),
        ],
    )
    out = pl.pallas_call(
        _moe_body,
        grid_spec=grid_spec,
        out_shape=jax.ShapeDtypeStruct((B, _C, _P), jnp.bfloat16),
        compiler_params=pltpu.CompilerParams(
            dimension_semantics=("parallel",)),
    )(idx, vals, xp, mask, W1t, W1t, b1r, b1r, W2t, W2t, b2r, b2r)

    out = out.reshape(B, _C, _HP, _WP)[:, :, 1:-1, 1:-1]
    return out.astype(jnp.float32)
